# Initial kernel scaffold; baseline (speedup 1.0000x reference)
import jax
import jax.numpy as jnp
from jax.experimental import pallas as pl


def kernel(token_id, num_mag, num_pre, num_top, num_low, order, pos_row, pos_col, pos_top, pos_left, format_vec, token_table, magnitude_table, precision_table, top_digit_table, low_digit_table, order_table, row_table, column_table, top_tree_table, left_tree_table, format_W, ln_gamma, ln_beta):
    raise NotImplementedError("write your pallas kernel here")


# speedup vs baseline: 7.6196x; 7.6196x over previous
import functools

import jax
import jax.numpy as jnp
from jax import lax
from jax.experimental import pallas as pl
from jax.experimental.pallas import tpu as pltpu
from jax.experimental.pallas import tpu_sc as plsc

_B, _S, _D = 128, 200, 768
_BS = _B * _S
_TB = 256
_GRID = _BS // _TB
_NW = 32
_BPW = _BS // _NW
_CH = 80
_NCH = _BPW // _CH
_EPS = 1e-12


def _sc_gather(table, idx):
    mesh = plsc.VectorSubcoreMesh(core_axis_name="c", subcore_axis_name="s")

    @functools.partial(
        pl.kernel,
        mesh=mesh,
        out_type=jax.ShapeDtypeStruct((_BS, _D), jnp.float32),
        scratch_types=[
            pltpu.VMEM((_BPW,), jnp.int32),
            pltpu.VMEM((_CH, _D), jnp.float32),
            pltpu.SemaphoreType.DMA,
        ],
    )
    def k(table_hbm, idx_hbm, out_hbm, idx_v, rows_v, sem):
        wid = lax.axis_index("s") * 2 + lax.axis_index("c")
        base = wid * _BPW
        pltpu.sync_copy(idx_hbm.at[pl.ds(base, _BPW)], idx_v)
        for c in range(_NCH):
            pltpu.async_copy(
                table_hbm.at[idx_v.at[pl.ds(c * _CH, _CH)]], rows_v, sem
            ).wait()
            pltpu.sync_copy(rows_v, out_hbm.at[pl.ds(base + c * _CH, _CH)])

    return k(table, idx)


def _tc_body(tok, magI, preI, topI, lowI, ordI, rowI, colI,
             lt0, lt1, lt2, lt3, tt0, tt1, tt2, tt3, fv,
             numT, ordT, rowT, colT, ltT, ttT, fWT, g, b, out):
    f32 = jnp.float32

    def dot(a, t):
        return lax.dot_general(a, t, (((1,), (0,)), ((), ())),
                               preferred_element_type=f32)

    def oh(iref, n):
        idx = iref[0, 0, :]
        io = lax.broadcasted_iota(jnp.int32, (_TB, n), 1)
        return jnp.where(io == idx[:, None], 1.0, 0.0).astype(jnp.bfloat16)

    io64 = lax.broadcasted_iota(jnp.int32, (_TB, 64), 1)
    sel = jnp.where(io64 < 16, magI[0, 0, :][:, None],
          jnp.where(io64 < 32, preI[0, 0, :][:, None] + 16,
          jnp.where(io64 < 48, topI[0, 0, :][:, None] + 32,
                    lowI[0, 0, :][:, None] + 48)))
    ohn = jnp.where(io64 == sel, 1.0, 0.0).astype(jnp.bfloat16)
    numeric = dot(ohn, numT[...])

    orders = dot(oh(ordI, 64), ordT[...])
    rows = dot(oh(rowI, 264), rowT[...])
    cols = dot(oh(colI, 264), colT[...])
    lts = [dot(oh(r, 392), ltT[...]) for r in (lt0, lt1, lt2, lt3)]
    tts = [dot(oh(r, 392), ttT[...]) for r in (tt0, tt1, tt2, tt3)]
    fmt = dot(fv[0], fWT[...])

    pos = jnp.concatenate([rows] + lts + [cols] + tts, axis=1)
    emb = tok[...] + numeric + orders + pos + fmt
    mu = jnp.mean(emb, axis=1, keepdims=True)
    cen = emb - mu
    var = jnp.mean(cen * cen, axis=1, keepdims=True)
    out[...] = cen * lax.rsqrt(var + _EPS) * g[...] + b[...]


def _tc_fused(tok, idxs, fv, tables, g, b):
    ispec = pl.BlockSpec((1, 1, _TB), lambda i: (i, 0, 0))

    def full(shape):
        r = len(shape)
        return pl.BlockSpec(shape, lambda i, _r=r: (0,) * _r)

    return pl.pallas_call(
        _tc_body,
        grid=(_GRID,),
        in_specs=[pl.BlockSpec((_TB, _D), lambda i: (i, 0))]
        + [ispec] * 15
        + [pl.BlockSpec((1, _TB, 16), lambda i: (i, 0, 0)),
           full((64, _D)), full((64, _D)),
           full((264, 96)), full((264, 96)),
           full((392, 72)), full((392, 72)),
           full((16, _D)), full((1, _D)), full((1, _D))],
        out_specs=pl.BlockSpec((_TB, _D), lambda i: (i, 0)),
        out_shape=jax.ShapeDtypeStruct((_BS, _D), jnp.float32),
    )(tok, *idxs, fv, *tables, g, b)


def kernel(token_id, num_mag, num_pre, num_top, num_low, order, pos_row,
           pos_col, pos_top, pos_left, format_vec, token_table,
           magnitude_table, precision_table, top_digit_table,
           low_digit_table, order_table, row_table, column_table,
           top_tree_table, left_tree_table, format_W, ln_gamma, ln_beta):
    bf16 = jnp.bfloat16

    tok = _sc_gather(token_table, token_id.reshape(_BS).astype(jnp.int32))

    def idx3(a):
        return a.reshape(_GRID, 1, _TB).astype(jnp.int32)

    pt = pos_top.reshape(_BS, 4)
    pf = pos_left.reshape(_BS, 4)
    idxs = [idx3(num_mag), idx3(num_pre), idx3(num_top), idx3(num_low),
            idx3(order), idx3(pos_row), idx3(pos_col),
            idx3(pf[:, 0]), idx3(pf[:, 1]), idx3(pf[:, 2]), idx3(pf[:, 3]),
            idx3(pt[:, 0]), idx3(pt[:, 1]), idx3(pt[:, 2]), idx3(pt[:, 3])]

    numT = jnp.zeros((64, _D), jnp.float32)
    numT = (numT.at[0:12, 0:192].set(magnitude_table)
                .at[16:28, 192:384].set(precision_table)
                .at[32:44, 384:576].set(top_digit_table)
                .at[48:60, 576:768].set(low_digit_table)).astype(bf16)
    ordT = order_table.astype(bf16)
    rowT = jnp.pad(row_table, ((0, 7), (0, 0))).astype(bf16)
    colT = jnp.pad(column_table, ((0, 7), (0, 0))).astype(bf16)
    ltT = jnp.pad(left_tree_table, ((0, 7), (0, 0))).astype(bf16)
    ttT = jnp.pad(top_tree_table, ((0, 7), (0, 0))).astype(bf16)
    fWT = jnp.pad(format_W.T, ((0, 5), (0, 0))).astype(bf16)
    fv = jnp.pad(format_vec.reshape(_BS, 11), ((0, 0), (0, 5))
                 ).astype(bf16).reshape(_GRID, _TB, 16)

    out = _tc_fused(tok, idxs, fv,
                    [numT, ordT, rowT, colT, ltT, ttT, fWT],
                    ln_gamma.reshape(1, _D), ln_beta.reshape(1, _D))
    return out.reshape(_B, _S, _D)


# speedup vs baseline: 8.2774x; 1.0863x over previous
import functools

import jax
import jax.numpy as jnp
from jax import lax
from jax.experimental import pallas as pl
from jax.experimental.pallas import tpu as pltpu
from jax.experimental.pallas import tpu_sc as plsc

_B, _S, _D = 128, 200, 768
_BS = _B * _S
_TB = 512
_GRID = _BS // _TB
_NW = 32
_BPW = _BS // _NW
_CH = 80
_NCH = _BPW // _CH
_EPS = 1e-12


def _sc_gather(table, idx):
    mesh = plsc.VectorSubcoreMesh(core_axis_name="c", subcore_axis_name="s")

    @functools.partial(
        pl.kernel,
        mesh=mesh,
        out_type=jax.ShapeDtypeStruct((_BS, _D), jnp.float32),
        scratch_types=[
            pltpu.VMEM((_BPW,), jnp.int32),
            pltpu.VMEM((_CH, _D), jnp.float32),
            pltpu.VMEM((_CH, _D), jnp.float32),
            pltpu.SemaphoreType.DMA,
            pltpu.SemaphoreType.DMA,
        ],
    )
    def k(table_hbm, idx_hbm, out_hbm, idx_v, rows_a, rows_b, sem_a, sem_b):
        wid = lax.axis_index("s") * 2 + lax.axis_index("c")
        base = wid * _BPW
        pltpu.sync_copy(idx_hbm.at[pl.ds(base, _BPW)], idx_v)
        bufs = (rows_a, rows_b)
        sems = (sem_a, sem_b)
        cps = [None, None]
        cps[0] = pltpu.async_copy(
            table_hbm.at[idx_v.at[pl.ds(0, _CH)]], bufs[0], sems[0])
        for c in range(_NCH):
            if c + 1 < _NCH:
                cps[(c + 1) % 2] = pltpu.async_copy(
                    table_hbm.at[idx_v.at[pl.ds((c + 1) * _CH, _CH)]],
                    bufs[(c + 1) % 2], sems[(c + 1) % 2])
            cps[c % 2].wait()
            pltpu.sync_copy(bufs[c % 2],
                            out_hbm.at[pl.ds(base + c * _CH, _CH)])

    return k(table, idx)


def _tc_body(tok, magI, preI, topI, lowI, ordI, rowI, colI,
             lt0, lt1, lt2, lt3, tt0, tt1, tt2, tt3, fv,
             numT, ordT, rowT, colT, ltT, ttT, fWT, g, b, out):
    f32 = jnp.float32

    def dot(a, t):
        return lax.dot_general(a, t, (((1,), (0,)), ((), ())),
                               preferred_element_type=f32)

    def oh(iref, n):
        idx = iref[0, 0, :]
        io = lax.broadcasted_iota(jnp.int32, (_TB, n), 1)
        return (io == idx[:, None]).astype(jnp.bfloat16)

    io64 = lax.broadcasted_iota(jnp.int32, (_TB, 64), 1)
    sel = jnp.where(io64 < 16, magI[0, 0, :][:, None],
          jnp.where(io64 < 32, preI[0, 0, :][:, None] + 16,
          jnp.where(io64 < 48, topI[0, 0, :][:, None] + 32,
                    lowI[0, 0, :][:, None] + 48)))
    ohn = (io64 == sel).astype(jnp.bfloat16)
    numeric = dot(ohn, numT[...])

    orders = dot(oh(ordI, 64), ordT[...])
    rows = dot(oh(rowI, 264), rowT[...])
    cols = dot(oh(colI, 264), colT[...])
    lts = [dot(oh(r, 392), ltT[...]) for r in (lt0, lt1, lt2, lt3)]
    tts = [dot(oh(r, 392), ttT[...]) for r in (tt0, tt1, tt2, tt3)]
    fmt = dot(fv[0], fWT[...])

    pos = jnp.concatenate([rows] + lts + [cols] + tts, axis=1)
    emb = tok[...] + numeric + orders + pos + fmt
    mu = jnp.mean(emb, axis=1, keepdims=True)
    cen = emb - mu
    var = jnp.mean(cen * cen, axis=1, keepdims=True)
    out[...] = cen * lax.rsqrt(var + _EPS) * g[...] + b[...]


def _tc_fused(tok, idxs, fv, tables, g, b):
    ispec = pl.BlockSpec((1, 1, _TB), lambda i: (i, 0, 0))

    def full(shape):
        r = len(shape)
        return pl.BlockSpec(shape, lambda i, _r=r: (0,) * _r)

    return pl.pallas_call(
        _tc_body,
        grid=(_GRID,),
        in_specs=[pl.BlockSpec((_TB, _D), lambda i: (i, 0))]
        + [ispec] * 15
        + [pl.BlockSpec((1, _TB, 16), lambda i: (i, 0, 0)),
           full((64, _D)), full((64, _D)),
           full((264, 96)), full((264, 96)),
           full((392, 72)), full((392, 72)),
           full((16, _D)), full((1, _D)), full((1, _D))],
        out_specs=pl.BlockSpec((_TB, _D), lambda i: (i, 0)),
        out_shape=jax.ShapeDtypeStruct((_BS, _D), jnp.float32),
    )(tok, *idxs, fv, *tables, g, b)


def kernel(token_id, num_mag, num_pre, num_top, num_low, order, pos_row,
           pos_col, pos_top, pos_left, format_vec, token_table,
           magnitude_table, precision_table, top_digit_table,
           low_digit_table, order_table, row_table, column_table,
           top_tree_table, left_tree_table, format_W, ln_gamma, ln_beta):
    bf16 = jnp.bfloat16

    tok = _sc_gather(token_table, token_id.reshape(_BS).astype(jnp.int32))

    def idx3(a):
        return a.reshape(_GRID, 1, _TB).astype(jnp.int32)

    pt = pos_top.reshape(_BS, 4)
    pf = pos_left.reshape(_BS, 4)
    idxs = [idx3(num_mag), idx3(num_pre), idx3(num_top), idx3(num_low),
            idx3(order), idx3(pos_row), idx3(pos_col),
            idx3(pf[:, 0]), idx3(pf[:, 1]), idx3(pf[:, 2]), idx3(pf[:, 3]),
            idx3(pt[:, 0]), idx3(pt[:, 1]), idx3(pt[:, 2]), idx3(pt[:, 3])]

    numT = jnp.zeros((64, _D), jnp.float32)
    numT = (numT.at[0:12, 0:192].set(magnitude_table)
                .at[16:28, 192:384].set(precision_table)
                .at[32:44, 384:576].set(top_digit_table)
                .at[48:60, 576:768].set(low_digit_table)).astype(bf16)
    ordT = order_table.astype(bf16)
    rowT = jnp.pad(row_table, ((0, 7), (0, 0))).astype(bf16)
    colT = jnp.pad(column_table, ((0, 7), (0, 0))).astype(bf16)
    ltT = jnp.pad(left_tree_table, ((0, 7), (0, 0))).astype(bf16)
    ttT = jnp.pad(top_tree_table, ((0, 7), (0, 0))).astype(bf16)
    fWT = jnp.pad(format_W.T, ((0, 5), (0, 0))).astype(bf16)
    fv = jnp.pad(format_vec.reshape(_BS, 11), ((0, 0), (0, 5))
                 ).astype(bf16).reshape(_GRID, _TB, 16)

    out = _tc_fused(tok, idxs, fv,
                    [numT, ordT, rowT, colT, ltT, ttT, fWT],
                    ln_gamma.reshape(1, _D), ln_beta.reshape(1, _D))
    return out.reshape(_B, _S, _D)
